# baseline (speedup 1.0000x reference)
import jax
import jax.numpy as jnp
from jax.experimental import pallas as pl
from jax.experimental.pallas import tpu as pltpu

N = 10000
BM = 400
NM = N // BM
NBUF = 3


def _gcn_kernel(x_ref, adj_hbm, w_ref, b_ref, out_ref, sup_ref, buf_ref, sem):
    def start_copy(blk, slot):
        pltpu.make_async_copy(
            adj_hbm.at[pl.ds(blk * BM, BM), :],
            buf_ref.at[slot],
            sem.at[slot],
        ).start()

    for j in range(NBUF):
        start_copy(j, j)

    sup_ref[:, :] = jnp.dot(
        x_ref[:, :], w_ref[:, :], preferred_element_type=jnp.float32
    )

    def body(i, _):
        slot = jax.lax.rem(i, NBUF)
        pltpu.make_async_copy(
            adj_hbm.at[pl.ds(i * BM, BM), :], buf_ref.at[slot], sem.at[slot]
        ).wait()
        h = jnp.dot(
            buf_ref[slot], sup_ref[:, :], preferred_element_type=jnp.float32
        )
        h = jax.nn.relu(h + b_ref[:, :])
        m = jnp.max(h, axis=1, keepdims=True)
        lse = jnp.log(jnp.sum(jnp.exp(h - m), axis=1, keepdims=True)) + m
        out_ref[pl.ds(i * BM, BM), :] = h - lse

        @pl.when(i + NBUF < NM)
        def _():
            start_copy(i + NBUF, slot)

        return 0

    jax.lax.fori_loop(0, NM, body, 0)


@jax.jit
def _run(x, adj, W, b):
    nhid = W.shape[1]
    return pl.pallas_call(
        _gcn_kernel,
        in_specs=[
            pl.BlockSpec(memory_space=pltpu.VMEM),
            pl.BlockSpec(memory_space=pl.ANY),
            pl.BlockSpec(memory_space=pltpu.VMEM),
            pl.BlockSpec(memory_space=pltpu.VMEM),
        ],
        out_specs=pl.BlockSpec(memory_space=pltpu.VMEM),
        out_shape=jax.ShapeDtypeStruct((N, nhid), jnp.float32),
        scratch_shapes=[
            pltpu.VMEM((N, nhid), jnp.float32),
            pltpu.VMEM((NBUF, BM, N), jnp.float32),
            pltpu.SemaphoreType.DMA((NBUF,)),
        ],
        compiler_params=pltpu.CompilerParams(
            vmem_limit_bytes=100 * 1024 * 1024,
        ),
    )(x, adj, W, b)


def kernel(x, adj, W, b):
    return _run(x, adj, W, b.reshape(1, -1))
